# Initial kernel scaffold; baseline (speedup 1.0000x reference)
import functools

import jax
import jax.numpy as jnp
from jax import lax
from jax.experimental import pallas as pl
from jax.experimental.pallas import tpu as pltpu
from jax.experimental.pallas import tpu_sc as plsc

NC = 2
NS = 16
NW = NC * NS
L = 16
CW = 16
LN_EPS = 1e-5


def _silu(x):
    return x / (1.0 + jnp.exp(-x))


def _sc_mesh():
    return plsc.VectorSubcoreMesh(
        core_axis_name="c", subcore_axis_name="s", num_cores=NC, num_subcores=NS
    )


def _pre_body(nf_ref, gamma_ref, beta_ref, w1a_ref, w1b_ref, p_ref, q_ref):
    x = nf_ref[...]
    mu = jnp.mean(x, axis=-1, keepdims=True)
    xc = x - mu
    var = jnp.mean(xc * xc, axis=-1, keepdims=True)
    h = xc * lax.rsqrt(var + LN_EPS) * gamma_ref[...] + beta_ref[...]
    p_ref[...] = jnp.dot(h, w1a_ref[...], preferred_element_type=jnp.float32)
    q_ref[...] = jnp.dot(h, w1b_ref[...], preferred_element_type=jnp.float32)


def _pre_tc(nf, gamma, beta, w1a, w1b):
    n, d = nf.shape
    bn = 1000
    return pl.pallas_call(
        _pre_body,
        grid=(n // bn,),
        in_specs=[
            pl.BlockSpec((bn, d), lambda i: (i, 0)),
            pl.BlockSpec((1, d), lambda i: (0, 0)),
            pl.BlockSpec((1, d), lambda i: (0, 0)),
            pl.BlockSpec((d, d), lambda i: (0, 0)),
            pl.BlockSpec((d, d), lambda i: (0, 0)),
        ],
        out_specs=[pl.BlockSpec((bn, d), lambda i: (i, 0))] * 2,
        out_shape=[jax.ShapeDtypeStruct((n, d), jnp.float32)] * 2,
    )(nf, gamma.reshape(1, d), beta.reshape(1, d), w1a, w1b)


def _gather_sc(p, q, src, dst):
    n, d = p.shape
    e = src.shape[0]
    ew = e // NW
    c = 80
    k = ew // c

    @functools.partial(
        pl.kernel,
        out_type=jax.ShapeDtypeStruct((e, d), jnp.float32),
        mesh=_sc_mesh(),
        scratch_types=[
            pltpu.VMEM((c,), jnp.int32),
            pltpu.VMEM((c,), jnp.int32),
            pltpu.VMEM((c, d), jnp.float32),
            pltpu.VMEM((c, d), jnp.float32),
            pltpu.SemaphoreType.DMA,
            pltpu.SemaphoreType.DMA,
        ],
    )
    def gather_kernel(p_hbm, q_hbm, src_hbm, dst_hbm, g_hbm,
                      idx_s, idx_d, bufp, bufq, semp, semq):
        wid = lax.axis_index("s") * NC + lax.axis_index("c")
        base = wid * ew

        def body(kk, carry):
            off = base + kk * c
            pltpu.sync_copy(src_hbm.at[pl.ds(off, c)], idx_s)
            pltpu.sync_copy(dst_hbm.at[pl.ds(off, c)], idx_d)
            cp1 = pltpu.async_copy(p_hbm.at[idx_s], bufp, semp)
            cp2 = pltpu.async_copy(q_hbm.at[idx_d], bufq, semq)
            cp1.wait()
            cp2.wait()

            def add_row(i, cc):
                for j in range(d // L):
                    sl = pl.ds(j * L, L)
                    bufp[i, sl] = bufp[i, sl] + bufq[i, sl]
                return cc

            lax.fori_loop(0, c, add_row, 0)
            pltpu.sync_copy(bufp, g_hbm.at[pl.ds(off, c)])
            return carry

        lax.fori_loop(0, k, body, 0)

    return gather_kernel(p, q, src, dst)


def _edge_body(g_ref, ef_ref, w1c_ref, b1_ref, w2_ref, b2_ref, out_ref):
    u = (g_ref[...]
         + jnp.dot(ef_ref[...], w1c_ref[...], preferred_element_type=jnp.float32)
         + b1_ref[...])
    u = _silu(u)
    v = jnp.dot(u, w2_ref[...], preferred_element_type=jnp.float32) + b2_ref[...]
    out_ref[...] = _silu(v)


def _edge_tc(g, ef, w1c, b1, w2, b2):
    e, d = g.shape
    ed = ef.shape[1]
    be = 2000
    return pl.pallas_call(
        _edge_body,
        grid=(e // be,),
        in_specs=[
            pl.BlockSpec((be, d), lambda i: (i, 0)),
            pl.BlockSpec((be, ed), lambda i: (i, 0)),
            pl.BlockSpec((ed, d), lambda i: (0, 0)),
            pl.BlockSpec((1, d), lambda i: (0, 0)),
            pl.BlockSpec((d, d), lambda i: (0, 0)),
            pl.BlockSpec((1, d), lambda i: (0, 0)),
        ],
        out_specs=pl.BlockSpec((be, d), lambda i: (i, 0)),
        out_shape=jax.ShapeDtypeStruct((e, d), jnp.float32),
    )(g, ef, w1c, b1.reshape(1, d), w2, b2.reshape(1, d))


def _scatter_sc(mij, src, n):
    e, d = mij.shape
    ew = e // NW
    c = 80
    k = ew // c
    sr = n // NS
    zr = 125
    zk = sr // zr

    @functools.partial(
        pl.kernel,
        out_type=[
            jax.ShapeDtypeStruct((NC, n, d), jnp.float32),
            jax.ShapeDtypeStruct((NC, n, CW), jnp.float32),
        ],
        mesh=_sc_mesh(),
        scratch_types=[
            pltpu.VMEM((c,), jnp.int32),
            pltpu.VMEM((c, d), jnp.float32),
            pltpu.VMEM((c, CW), jnp.float32),
            pltpu.VMEM((zr, d), jnp.float32),
            pltpu.VMEM((zr, CW), jnp.float32),
            pltpu.VMEM_SHARED((n, d), jnp.float32),
            pltpu.VMEM_SHARED((n, CW), jnp.float32),
        ],
    )
    def scatter_kernel(mij_hbm, src_hbm, outsum_hbm, outcnt_hbm,
                       idx, buf, ones, zbuf, zbufc, accum, cnt):
        cid = lax.axis_index("c")
        sid = lax.axis_index("s")
        base = (sid * NC + cid) * ew
        row0 = sid * sr

        def z_row(i, cc):
            for j in range(d // L):
                zbuf[i, pl.ds(j * L, L)] = jnp.zeros((L,), jnp.float32)
            zbufc[i, pl.ds(0, L)] = jnp.zeros((L,), jnp.float32)
            return cc

        lax.fori_loop(0, zr, z_row, 0)

        def o_row(i, cc):
            ones[i, pl.ds(0, L)] = jnp.ones((L,), jnp.float32)
            return cc

        lax.fori_loop(0, c, o_row, 0)

        def zcp(z, cc):
            r = row0 + z * zr
            pltpu.sync_copy(zbuf, accum.at[pl.ds(r, zr)])
            pltpu.sync_copy(zbufc, cnt.at[pl.ds(r, zr)])
            return cc

        lax.fori_loop(0, zk, zcp, 0)
        plsc.subcore_barrier()

        def body(kk, cc):
            off = base + kk * c
            pltpu.sync_copy(src_hbm.at[pl.ds(off, c)], idx)
            pltpu.sync_copy(mij_hbm.at[pl.ds(off, c)], buf)
            pltpu.sync_copy(buf, accum.at[idx], add=True)
            pltpu.sync_copy(ones, cnt.at[idx], add=True)
            return cc

        lax.fori_loop(0, k, body, 0)
        plsc.subcore_barrier()

        def wb(z, cc):
            r = row0 + z * zr
            pltpu.sync_copy(accum.at[pl.ds(r, zr)], zbuf)
            pltpu.sync_copy(zbuf, outsum_hbm.at[cid, pl.ds(r, zr)])
            pltpu.sync_copy(cnt.at[pl.ds(r, zr)], zbufc)
            pltpu.sync_copy(zbufc, outcnt_hbm.at[cid, pl.ds(r, zr)])
            return cc

        lax.fori_loop(0, zk, wb, 0)

    return scatter_kernel(mij, src)


def _final_body(nf_ref, ms0_ref, ms1_ref, cn0_ref, cn1_ref,
                a1a_ref, a1b_ref, c1_ref, a2_ref, c2_ref, out_ref):
    x = nf_ref[...]
    cnt = cn0_ref[...][:, :1] + cn1_ref[...][:, :1]
    msg = (ms0_ref[...] + ms1_ref[...]) / jnp.maximum(cnt, 1.0)
    u = (jnp.dot(x, a1a_ref[...], preferred_element_type=jnp.float32)
         + jnp.dot(msg, a1b_ref[...], preferred_element_type=jnp.float32)
         + c1_ref[...])
    u = _silu(u)
    v = jnp.dot(u, a2_ref[...], preferred_element_type=jnp.float32) + c2_ref[...]
    out_ref[...] = x + _silu(v)


def _final_tc(nf, ms0, ms1, cn0, cn1, a1a, a1b, c1, a2, c2):
    n, d = nf.shape
    bn = 1000
    return pl.pallas_call(
        _final_body,
        grid=(n // bn,),
        in_specs=[
            pl.BlockSpec((bn, d), lambda i: (i, 0)),
            pl.BlockSpec((bn, d), lambda i: (i, 0)),
            pl.BlockSpec((bn, d), lambda i: (i, 0)),
            pl.BlockSpec((bn, CW), lambda i: (i, 0)),
            pl.BlockSpec((bn, CW), lambda i: (i, 0)),
            pl.BlockSpec((d, d), lambda i: (0, 0)),
            pl.BlockSpec((d, d), lambda i: (0, 0)),
            pl.BlockSpec((1, d), lambda i: (0, 0)),
            pl.BlockSpec((d, d), lambda i: (0, 0)),
            pl.BlockSpec((1, d), lambda i: (0, 0)),
        ],
        out_specs=pl.BlockSpec((bn, d), lambda i: (i, 0)),
        out_shape=jax.ShapeDtypeStruct((n, d), jnp.float32),
    )(nf, ms0, ms1, cn0, cn1, a1a, a1b, c1.reshape(1, d), a2, c2.reshape(1, d))


def kernel(node_features, edge_features, edge_index, edge2graph,
           W1, b1, W2, b2, A1, c1, A2, c2, gamma, beta):
    n, d = node_features.shape
    src = edge_index[0]
    dst = edge_index[1]
    w1a, w1b, w1c = W1[:d], W1[d:2 * d], W1[2 * d:]
    a1a, a1b = A1[:d], A1[d:]

    p, q = _pre_tc(node_features, gamma, beta, w1a, w1b)
    g = _gather_sc(p, q, src, dst)
    mij = _edge_tc(g, edge_features, w1c, b1, W2, b2)
    msum, cnt = _scatter_sc(mij, src, n)
    return _final_tc(node_features, msum[0], msum[1], cnt[0], cnt[1],
                     a1a, a1b, c1, A2, c2)


# speedup vs baseline: 3.0734x; 3.0734x over previous
import functools

import jax
import jax.numpy as jnp
from jax import lax
from jax.experimental import pallas as pl
from jax.experimental.pallas import tpu as pltpu
from jax.experimental.pallas import tpu_sc as plsc

NC = 2
NS = 16
NW = NC * NS
L = 16
LN_EPS = 1e-5


def _silu(x):
    return x / (1.0 + jnp.exp(-x))


def _sc_mesh():
    return plsc.VectorSubcoreMesh(
        core_axis_name="c", subcore_axis_name="s", num_cores=NC, num_subcores=NS
    )


def _pre_body(nf_ref, gamma_ref, beta_ref, w1a_ref, w1b_ref, p_ref, q_ref):
    x = nf_ref[...]
    mu = jnp.mean(x, axis=-1, keepdims=True)
    xc = x - mu
    var = jnp.mean(xc * xc, axis=-1, keepdims=True)
    h = xc * lax.rsqrt(var + LN_EPS) * gamma_ref[...] + beta_ref[...]
    p_ref[...] = jnp.dot(h, w1a_ref[...], preferred_element_type=jnp.float32)
    q_ref[...] = jnp.dot(h, w1b_ref[...], preferred_element_type=jnp.float32)


def _pre_tc(nf, gamma, beta, w1a, w1b):
    n, d = nf.shape
    bn = 1000
    return pl.pallas_call(
        _pre_body,
        grid=(n // bn,),
        in_specs=[
            pl.BlockSpec((bn, d), lambda i: (i, 0)),
            pl.BlockSpec((1, d), lambda i: (0, 0)),
            pl.BlockSpec((1, d), lambda i: (0, 0)),
            pl.BlockSpec((d, d), lambda i: (0, 0)),
            pl.BlockSpec((d, d), lambda i: (0, 0)),
        ],
        out_specs=[pl.BlockSpec((bn, d), lambda i: (i, 0))] * 2,
        out_shape=[jax.ShapeDtypeStruct((n, d), jnp.float32)] * 2,
    )(nf, gamma.reshape(1, d), beta.reshape(1, d), w1a, w1b)


def _gather_sc(p, q, src, dst):
    n, d = p.shape
    e = src.shape[0]
    ew = e // NW
    c = 80
    k = ew // c
    sr = (n // NS) // 8 * 8
    zr = 16
    tail = n - NS * sr

    @functools.partial(
        pl.kernel,
        out_type=[
            jax.ShapeDtypeStruct((e, d), jnp.float32),
            jax.ShapeDtypeStruct((NC, n, d), jnp.float32),
        ],
        mesh=_sc_mesh(),
        scratch_types=[
            pltpu.VMEM((c,), jnp.int32),
            pltpu.VMEM((c,), jnp.int32),
            pltpu.VMEM((c, d), jnp.float32),
            pltpu.VMEM((c, d), jnp.float32),
            pltpu.VMEM((c, d), jnp.float32),
            pltpu.VMEM((zr, d), jnp.float32),
            pltpu.VMEM_SHARED((n, d), jnp.float32),
            pltpu.SemaphoreType.DMA,
            pltpu.SemaphoreType.DMA,
        ],
    )
    def gather_kernel(p_hbm, q_hbm, src_hbm, dst_hbm, g_hbm, outcnt_hbm,
                      idx_s, idx_d, bufp, bufq, ones, zbuf, cnt, semp, semq):
        cid = lax.axis_index("c")
        sid = lax.axis_index("s")
        wid = sid * NC + cid
        base = wid * ew
        row0 = sid * sr
        zk = sr // zr

        def fill_ones(i, cc):
            for j in range(d // L):
                ones[i, pl.ds(j * L, L)] = jnp.ones((L,), jnp.float32)
            return cc

        lax.fori_loop(0, c, fill_ones, 0)

        def fill_zero(i, cc):
            for j in range(d // L):
                zbuf[i, pl.ds(j * L, L)] = jnp.zeros((L,), jnp.float32)
            return cc

        lax.fori_loop(0, zr, fill_zero, 0)

        def zcp(z, cc):
            pltpu.sync_copy(zbuf, cnt.at[pl.ds(row0 + z * zr, zr)])
            return cc

        lax.fori_loop(0, zk, zcp, 0)

        @pl.when(sid == 0)
        def _zero_tail():
            def zcp_t(z, cc):
                pltpu.sync_copy(zbuf, cnt.at[pl.ds(NS * sr + z * zr, zr)])
                return cc
            lax.fori_loop(0, tail // zr, zcp_t, 0)

        plsc.subcore_barrier()

        def body(kk, carry):
            off = base + kk * c
            pltpu.sync_copy(src_hbm.at[pl.ds(off, c)], idx_s)
            pltpu.sync_copy(dst_hbm.at[pl.ds(off, c)], idx_d)
            cp1 = pltpu.async_copy(p_hbm.at[idx_s], bufp, semp)
            cp2 = pltpu.async_copy(q_hbm.at[idx_d], bufq, semq)
            pltpu.sync_copy(ones, cnt.at[idx_s], add=True)
            cp1.wait()
            cp2.wait()

            def add_row(i, cc):
                for j in range(d // L):
                    sl = pl.ds(j * L, L)
                    bufp[i, sl] = bufp[i, sl] + bufq[i, sl]
                return cc

            lax.fori_loop(0, c, add_row, 0)
            pltpu.sync_copy(bufp, g_hbm.at[pl.ds(off, c)])
            return carry

        lax.fori_loop(0, k, body, 0)
        plsc.subcore_barrier()

        def wb(z, cc):
            r = row0 + z * zr
            pltpu.sync_copy(cnt.at[pl.ds(r, zr)], zbuf)
            pltpu.sync_copy(zbuf, outcnt_hbm.at[cid, pl.ds(r, zr)])
            return cc

        lax.fori_loop(0, zk, wb, 0)

        @pl.when(sid == 0)
        def _wb_tail():
            def wb_t(z, cc):
                r = NS * sr + z * zr
                pltpu.sync_copy(cnt.at[pl.ds(r, zr)], zbuf)
                pltpu.sync_copy(zbuf, outcnt_hbm.at[cid, pl.ds(r, zr)])
                return cc
            lax.fori_loop(0, tail // zr, wb_t, 0)

    return gather_kernel(p, q, src, dst)


def _edge_body(g_ref, ef_ref, w1c_ref, b1_ref, w2_ref, b2_ref, out_ref):
    u = (g_ref[...]
         + jnp.dot(ef_ref[...], w1c_ref[...], preferred_element_type=jnp.float32)
         + b1_ref[...])
    u = _silu(u)
    v = jnp.dot(u, w2_ref[...], preferred_element_type=jnp.float32) + b2_ref[...]
    out_ref[...] = _silu(v)


def _edge_tc(g, ef, w1c, b1, w2, b2):
    e, d = g.shape
    ed = ef.shape[1]
    be = 2000
    return pl.pallas_call(
        _edge_body,
        grid=(e // be,),
        in_specs=[
            pl.BlockSpec((be, d), lambda i: (i, 0)),
            pl.BlockSpec((be, ed), lambda i: (i, 0)),
            pl.BlockSpec((ed, d), lambda i: (0, 0)),
            pl.BlockSpec((1, d), lambda i: (0, 0)),
            pl.BlockSpec((d, d), lambda i: (0, 0)),
            pl.BlockSpec((1, d), lambda i: (0, 0)),
        ],
        out_specs=pl.BlockSpec((be, d), lambda i: (i, 0)),
        out_shape=jax.ShapeDtypeStruct((e, d), jnp.float32),
    )(g, ef, w1c, b1.reshape(1, d), w2, b2.reshape(1, d))


def _scatter_sc(mij, src, n):
    e, d = mij.shape
    ew = e // NW
    c = 80
    k = ew // c
    sr = (n // NS) // 8 * 8
    zr = 16
    tail = n - NS * sr

    @functools.partial(
        pl.kernel,
        out_type=jax.ShapeDtypeStruct((NC, n, d), jnp.float32),
        mesh=_sc_mesh(),
        scratch_types=[
            pltpu.VMEM((c,), jnp.int32),
            pltpu.VMEM((c, d), jnp.float32),
            pltpu.VMEM((zr, d), jnp.float32),
            pltpu.VMEM_SHARED((n, d), jnp.float32),
        ],
    )
    def scatter_kernel(mij_hbm, src_hbm, outsum_hbm, idx, buf, zbuf, accum):
        cid = lax.axis_index("c")
        sid = lax.axis_index("s")
        base = (sid * NC + cid) * ew
        row0 = sid * sr
        zk = sr // zr

        def z_row(i, cc):
            for j in range(d // L):
                zbuf[i, pl.ds(j * L, L)] = jnp.zeros((L,), jnp.float32)
            return cc

        lax.fori_loop(0, zr, z_row, 0)

        def zcp(z, cc):
            pltpu.sync_copy(zbuf, accum.at[pl.ds(row0 + z * zr, zr)])
            return cc

        lax.fori_loop(0, zk, zcp, 0)

        @pl.when(sid == 0)
        def _zero_tail():
            def zcp_t(z, cc):
                pltpu.sync_copy(zbuf, accum.at[pl.ds(NS * sr + z * zr, zr)])
                return cc
            lax.fori_loop(0, tail // zr, zcp_t, 0)

        plsc.subcore_barrier()

        def body(kk, cc):
            off = base + kk * c
            pltpu.sync_copy(src_hbm.at[pl.ds(off, c)], idx)
            pltpu.sync_copy(mij_hbm.at[pl.ds(off, c)], buf)
            pltpu.sync_copy(buf, accum.at[idx], add=True)
            return cc

        lax.fori_loop(0, k, body, 0)
        plsc.subcore_barrier()

        def wb(z, cc):
            r = row0 + z * zr
            pltpu.sync_copy(accum.at[pl.ds(r, zr)], zbuf)
            pltpu.sync_copy(zbuf, outsum_hbm.at[cid, pl.ds(r, zr)])
            return cc

        lax.fori_loop(0, zk, wb, 0)

        @pl.when(sid == 0)
        def _wb_tail():
            def wb_t(z, cc):
                r = NS * sr + z * zr
                pltpu.sync_copy(accum.at[pl.ds(r, zr)], zbuf)
                pltpu.sync_copy(zbuf, outsum_hbm.at[cid, pl.ds(r, zr)])
                return cc
            lax.fori_loop(0, tail // zr, wb_t, 0)

    return scatter_kernel(mij, src)


def _final_body(nf_ref, ms0_ref, ms1_ref, cn0_ref, cn1_ref,
                a1a_ref, a1b_ref, c1_ref, a2_ref, c2_ref, out_ref):
    x = nf_ref[...]
    cnt = cn0_ref[...][:, :1] + cn1_ref[...][:, :1]
    msg = (ms0_ref[...] + ms1_ref[...]) / jnp.maximum(cnt, 1.0)
    u = (jnp.dot(x, a1a_ref[...], preferred_element_type=jnp.float32)
         + jnp.dot(msg, a1b_ref[...], preferred_element_type=jnp.float32)
         + c1_ref[...])
    u = _silu(u)
    v = jnp.dot(u, a2_ref[...], preferred_element_type=jnp.float32) + c2_ref[...]
    out_ref[...] = x + _silu(v)


def _final_tc(nf, ms0, ms1, cn0, cn1, a1a, a1b, c1, a2, c2):
    n, d = nf.shape
    bn = 1000
    return pl.pallas_call(
        _final_body,
        grid=(n // bn,),
        in_specs=[pl.BlockSpec((bn, d), lambda i: (i, 0))] * 5 + [
            pl.BlockSpec((d, d), lambda i: (0, 0)),
            pl.BlockSpec((d, d), lambda i: (0, 0)),
            pl.BlockSpec((1, d), lambda i: (0, 0)),
            pl.BlockSpec((d, d), lambda i: (0, 0)),
            pl.BlockSpec((1, d), lambda i: (0, 0)),
        ],
        out_specs=pl.BlockSpec((bn, d), lambda i: (i, 0)),
        out_shape=jax.ShapeDtypeStruct((n, d), jnp.float32),
    )(nf, ms0, ms1, cn0, cn1, a1a, a1b, c1.reshape(1, d), a2, c2.reshape(1, d))


def kernel(node_features, edge_features, edge_index, edge2graph,
           W1, b1, W2, b2, A1, c1, A2, c2, gamma, beta):
    n, d = node_features.shape
    src = edge_index[0]
    dst = edge_index[1]
    w1a, w1b, w1c = W1[:d], W1[d:2 * d], W1[2 * d:]
    a1a, a1b = A1[:d], A1[d:]

    p, q = _pre_tc(node_features, gamma, beta, w1a, w1b)
    g, cnth = _gather_sc(p, q, src, dst)
    mij = _edge_tc(g, edge_features, w1c, b1, W2, b2)
    msum = _scatter_sc(mij, src, n)
    return _final_tc(node_features, msum[0], msum[1], cnth[0], cnth[1],
                     a1a, a1b, c1, A2, c2)


# speedup vs baseline: 3.5633x; 1.1594x over previous
import functools

import jax
import jax.numpy as jnp
from jax import lax
from jax.experimental import pallas as pl
from jax.experimental.pallas import tpu as pltpu
from jax.experimental.pallas import tpu_sc as plsc

NC = 2
NS = 16
NW = NC * NS
L = 16
CW = 128
LN_EPS = 1e-5


def _silu(x):
    return x / (1.0 + jnp.exp(-x))


def _sc_mesh():
    return plsc.VectorSubcoreMesh(
        core_axis_name="c", subcore_axis_name="s", num_cores=NC, num_subcores=NS
    )


def _pre_body(nf_ref, gamma_ref, beta_ref, w1a_ref, w1b_ref, p_ref, q_ref):
    x = nf_ref[...]
    mu = jnp.mean(x, axis=-1, keepdims=True)
    xc = x - mu
    var = jnp.mean(xc * xc, axis=-1, keepdims=True)
    h = xc * lax.rsqrt(var + LN_EPS) * gamma_ref[...] + beta_ref[...]
    p_ref[...] = jnp.dot(h, w1a_ref[...], preferred_element_type=jnp.float32)
    q_ref[...] = jnp.dot(h, w1b_ref[...], preferred_element_type=jnp.float32)


def _pre_tc(nf, gamma, beta, w1a, w1b):
    n, d = nf.shape
    bn = 1000
    return pl.pallas_call(
        _pre_body,
        grid=(n // bn,),
        in_specs=[
            pl.BlockSpec((bn, d), lambda i: (i, 0)),
            pl.BlockSpec((1, d), lambda i: (0, 0)),
            pl.BlockSpec((1, d), lambda i: (0, 0)),
            pl.BlockSpec((d, d), lambda i: (0, 0)),
            pl.BlockSpec((d, d), lambda i: (0, 0)),
        ],
        out_specs=[pl.BlockSpec((bn, d), lambda i: (i, 0))] * 2,
        out_shape=[jax.ShapeDtypeStruct((n, d), jnp.float32)] * 2,
    )(nf, gamma.reshape(1, d), beta.reshape(1, d), w1a, w1b)


def _gather_sc(p, q, src, dst):
    n, d = p.shape
    e = src.shape[0]
    ew = e // NW
    c = 40
    k = ew // c
    sr = (n // NS) // 8 * 8
    zr = 16
    tail = n - NS * sr

    @functools.partial(
        pl.kernel,
        out_type=[
            jax.ShapeDtypeStruct((e, d), jnp.float32),
            jax.ShapeDtypeStruct((NC, n, CW), jnp.float32),
        ],
        mesh=_sc_mesh(),
        scratch_types=[
            pltpu.VMEM((c,), jnp.int32),
            pltpu.VMEM((c,), jnp.int32),
            pltpu.VMEM((c,), jnp.int32),
            pltpu.VMEM((c,), jnp.int32),
            pltpu.VMEM((c, d), jnp.float32),
            pltpu.VMEM((c, d), jnp.float32),
            pltpu.VMEM((c, d), jnp.float32),
            pltpu.VMEM((c, d), jnp.float32),
            pltpu.VMEM((c, CW), jnp.float32),
            pltpu.VMEM((zr, CW), jnp.float32),
            pltpu.VMEM_SHARED((n, CW), jnp.float32),
            pltpu.SemaphoreType.DMA,
            pltpu.SemaphoreType.DMA,
            pltpu.SemaphoreType.DMA,
            pltpu.SemaphoreType.DMA,
        ],
    )
    def gather_kernel(p_hbm, q_hbm, src_hbm, dst_hbm, g_hbm, outcnt_hbm,
                      idx_s0, idx_d0, idx_s1, idx_d1,
                      bufp0, bufq0, bufp1, bufq1,
                      ones, zbuf, cnt,
                      semp0, semq0, semp1, semq1):
        cid = lax.axis_index("c")
        sid = lax.axis_index("s")
        wid = sid * NC + cid
        base = wid * ew
        row0 = sid * sr
        zk = sr // zr

        sets = (
            (idx_s0, idx_d0, bufp0, bufq0, semp0, semq0),
            (idx_s1, idx_d1, bufp1, bufq1, semp1, semq1),
        )

        def fill_ones(i, cc):
            for j in range(CW // L):
                ones[i, pl.ds(j * L, L)] = jnp.ones((L,), jnp.float32)
            return cc

        lax.fori_loop(0, c, fill_ones, 0)

        def fill_zero(i, cc):
            for j in range(CW // L):
                zbuf[i, pl.ds(j * L, L)] = jnp.zeros((L,), jnp.float32)
            return cc

        lax.fori_loop(0, zr, fill_zero, 0)

        def zcp(z, cc):
            pltpu.sync_copy(zbuf, cnt.at[pl.ds(row0 + z * zr, zr)])
            return cc

        lax.fori_loop(0, zk, zcp, 0)

        @pl.when(sid == 0)
        def _zero_tail():
            def zcp_t(z, cc):
                pltpu.sync_copy(zbuf, cnt.at[pl.ds(NS * sr + z * zr, zr)])
                return cc
            lax.fori_loop(0, tail // zr, zcp_t, 0)

        plsc.subcore_barrier()

        def issue(kk, st):
            idx_s, idx_d, bufp, bufq, semp, semq = st
            off = base + kk * c
            pltpu.sync_copy(src_hbm.at[pl.ds(off, c)], idx_s)
            pltpu.sync_copy(dst_hbm.at[pl.ds(off, c)], idx_d)
            pltpu.async_copy(p_hbm.at[idx_s], bufp, semp)
            pltpu.async_copy(q_hbm.at[idx_d], bufq, semq)
            pltpu.sync_copy(ones, cnt.at[idx_s], add=True)

        def process(kk, st):
            idx_s, idx_d, bufp, bufq, semp, semq = st
            off = base + kk * c
            pltpu.make_async_copy(p_hbm.at[idx_s], bufp, semp).wait()
            pltpu.make_async_copy(q_hbm.at[idx_d], bufq, semq).wait()

            def add_row(i, cc):
                for j in range(d // L):
                    sl = pl.ds(j * L, L)
                    bufp[i, sl] = bufp[i, sl] + bufq[i, sl]
                return cc

            lax.fori_loop(0, c, add_row, 0)
            pltpu.sync_copy(bufp, g_hbm.at[pl.ds(off, c)])

        issue(0, sets[0])

        def body(t, cc):
            kk = 1 + 2 * t
            issue(kk, sets[1])
            process(kk - 1, sets[0])
            issue(kk + 1, sets[0])
            process(kk, sets[1])
            return cc

        lax.fori_loop(0, k // 2 - 1, body, 0)
        issue(k - 1, sets[1])
        process(k - 2, sets[0])
        process(k - 1, sets[1])
        plsc.subcore_barrier()

        def wb(z, cc):
            r = row0 + z * zr
            pltpu.sync_copy(cnt.at[pl.ds(r, zr)], zbuf)
            pltpu.sync_copy(zbuf, outcnt_hbm.at[cid, pl.ds(r, zr)])
            return cc

        lax.fori_loop(0, zk, wb, 0)

        @pl.when(sid == 0)
        def _wb_tail():
            def wb_t(z, cc):
                r = NS * sr + z * zr
                pltpu.sync_copy(cnt.at[pl.ds(r, zr)], zbuf)
                pltpu.sync_copy(zbuf, outcnt_hbm.at[cid, pl.ds(r, zr)])
                return cc
            lax.fori_loop(0, tail // zr, wb_t, 0)

    return gather_kernel(p, q, src, dst)


def _edge_body(g_ref, ef_ref, w1c_ref, b1_ref, w2_ref, b2_ref, out_ref):
    u = (g_ref[...]
         + jnp.dot(ef_ref[...], w1c_ref[...], preferred_element_type=jnp.float32)
         + b1_ref[...])
    u = _silu(u)
    v = jnp.dot(u, w2_ref[...], preferred_element_type=jnp.float32) + b2_ref[...]
    out_ref[...] = _silu(v)


def _edge_tc(g, ef, w1c, b1, w2, b2):
    e, d = g.shape
    ed = ef.shape[1]
    be = 2000
    return pl.pallas_call(
        _edge_body,
        grid=(e // be,),
        in_specs=[
            pl.BlockSpec((be, d), lambda i: (i, 0)),
            pl.BlockSpec((be, ed), lambda i: (i, 0)),
            pl.BlockSpec((ed, d), lambda i: (0, 0)),
            pl.BlockSpec((1, d), lambda i: (0, 0)),
            pl.BlockSpec((d, d), lambda i: (0, 0)),
            pl.BlockSpec((1, d), lambda i: (0, 0)),
        ],
        out_specs=pl.BlockSpec((be, d), lambda i: (i, 0)),
        out_shape=jax.ShapeDtypeStruct((e, d), jnp.float32),
    )(g, ef, w1c, b1.reshape(1, d), w2, b2.reshape(1, d))


def _scatter_sc(mij, src, n):
    e, d = mij.shape
    ew = e // NW
    c = 80
    k = ew // c
    sr = (n // NS) // 8 * 8
    zr = 16
    tail = n - NS * sr

    @functools.partial(
        pl.kernel,
        out_type=jax.ShapeDtypeStruct((NC, n, d), jnp.float32),
        mesh=_sc_mesh(),
        scratch_types=[
            pltpu.VMEM((c,), jnp.int32),
            pltpu.VMEM((c,), jnp.int32),
            pltpu.VMEM((c, d), jnp.float32),
            pltpu.VMEM((c, d), jnp.float32),
            pltpu.VMEM((zr, d), jnp.float32),
            pltpu.VMEM_SHARED((n, d), jnp.float32),
            pltpu.SemaphoreType.DMA,
            pltpu.SemaphoreType.DMA,
            pltpu.SemaphoreType.DMA,
            pltpu.SemaphoreType.DMA,
        ],
    )
    def scatter_kernel(mij_hbm, src_hbm, outsum_hbm,
                       idx0, idx1, buf0, buf1, zbuf, accum,
                       semr0, semi0, semr1, semi1):
        cid = lax.axis_index("c")
        sid = lax.axis_index("s")
        base = (sid * NC + cid) * ew
        row0 = sid * sr
        zk = sr // zr

        sets = ((idx0, buf0, semr0, semi0), (idx1, buf1, semr1, semi1))

        def z_row(i, cc):
            for j in range(d // L):
                zbuf[i, pl.ds(j * L, L)] = jnp.zeros((L,), jnp.float32)
            return cc

        lax.fori_loop(0, zr, z_row, 0)

        def zcp(z, cc):
            pltpu.sync_copy(zbuf, accum.at[pl.ds(row0 + z * zr, zr)])
            return cc

        lax.fori_loop(0, zk, zcp, 0)

        @pl.when(sid == 0)
        def _zero_tail():
            def zcp_t(z, cc):
                pltpu.sync_copy(zbuf, accum.at[pl.ds(NS * sr + z * zr, zr)])
                return cc
            lax.fori_loop(0, tail // zr, zcp_t, 0)

        plsc.subcore_barrier()

        def load(kk, st):
            idx, buf, semr, semi = st
            off = base + kk * c
            pltpu.async_copy(src_hbm.at[pl.ds(off, c)], idx, semi)
            pltpu.async_copy(mij_hbm.at[pl.ds(off, c)], buf, semr)

        def flush(kk, st):
            idx, buf, semr, semi = st
            off = base + kk * c
            pltpu.make_async_copy(src_hbm.at[pl.ds(off, c)], idx, semi).wait()
            pltpu.make_async_copy(mij_hbm.at[pl.ds(off, c)], buf, semr).wait()
            pltpu.sync_copy(buf, accum.at[idx], add=True)

        load(0, sets[0])

        def body(t, cc):
            kk = 1 + 2 * t
            load(kk, sets[1])
            flush(kk - 1, sets[0])
            load(kk + 1, sets[0])
            flush(kk, sets[1])
            return cc

        lax.fori_loop(0, (k - 1) // 2, body, 0)
        flush(k - 1, sets[0])
        plsc.subcore_barrier()

        def wb(z, cc):
            r = row0 + z * zr
            pltpu.sync_copy(accum.at[pl.ds(r, zr)], zbuf)
            pltpu.sync_copy(zbuf, outsum_hbm.at[cid, pl.ds(r, zr)])
            return cc

        lax.fori_loop(0, zk, wb, 0)

        @pl.when(sid == 0)
        def _wb_tail():
            def wb_t(z, cc):
                r = NS * sr + z * zr
                pltpu.sync_copy(accum.at[pl.ds(r, zr)], zbuf)
                pltpu.sync_copy(zbuf, outsum_hbm.at[cid, pl.ds(r, zr)])
                return cc
            lax.fori_loop(0, tail // zr, wb_t, 0)

    return scatter_kernel(mij, src)


def _final_body(nf_ref, ms0_ref, ms1_ref, cn0_ref, cn1_ref,
                a1a_ref, a1b_ref, c1_ref, a2_ref, c2_ref, out_ref):
    x = nf_ref[...]
    cnt = cn0_ref[...][:, :1] + cn1_ref[...][:, :1]
    msg = (ms0_ref[...] + ms1_ref[...]) / jnp.maximum(cnt, 1.0)
    u = (jnp.dot(x, a1a_ref[...], preferred_element_type=jnp.float32)
         + jnp.dot(msg, a1b_ref[...], preferred_element_type=jnp.float32)
         + c1_ref[...])
    u = _silu(u)
    v = jnp.dot(u, a2_ref[...], preferred_element_type=jnp.float32) + c2_ref[...]
    out_ref[...] = x + _silu(v)


def _final_tc(nf, ms0, ms1, cn0, cn1, a1a, a1b, c1, a2, c2):
    n, d = nf.shape
    bn = 1000
    return pl.pallas_call(
        _final_body,
        grid=(n // bn,),
        in_specs=[pl.BlockSpec((bn, d), lambda i: (i, 0))] * 3 + [
            pl.BlockSpec((bn, CW), lambda i: (i, 0)),
            pl.BlockSpec((bn, CW), lambda i: (i, 0)),
        ] + [
            pl.BlockSpec((d, d), lambda i: (0, 0)),
            pl.BlockSpec((d, d), lambda i: (0, 0)),
            pl.BlockSpec((1, d), lambda i: (0, 0)),
            pl.BlockSpec((d, d), lambda i: (0, 0)),
            pl.BlockSpec((1, d), lambda i: (0, 0)),
        ],
        out_specs=pl.BlockSpec((bn, d), lambda i: (i, 0)),
        out_shape=jax.ShapeDtypeStruct((n, d), jnp.float32),
    )(nf, ms0, ms1, cn0, cn1, a1a, a1b, c1.reshape(1, d), a2, c2.reshape(1, d))


def kernel(node_features, edge_features, edge_index, edge2graph,
           W1, b1, W2, b2, A1, c1, A2, c2, gamma, beta):
    n, d = node_features.shape
    src = edge_index[0]
    dst = edge_index[1]
    w1a, w1b, w1c = W1[:d], W1[d:2 * d], W1[2 * d:]
    a1a, a1b = A1[:d], A1[d:]

    p, q = _pre_tc(node_features, gamma, beta, w1a, w1b)
    g, cnth = _gather_sc(p, q, src, dst)
    mij = _edge_tc(g, edge_features, w1c, b1, W2, b2)
    msum = _scatter_sc(mij, src, n)
    return _final_tc(node_features, msum[0], msum[1], cnth[0], cnth[1],
                     a1a, a1b, c1, A2, c2)


# speedup vs baseline: 3.8015x; 1.0668x over previous
import functools

import jax
import jax.numpy as jnp
from jax import lax
from jax.experimental import pallas as pl
from jax.experimental.pallas import tpu as pltpu
from jax.experimental.pallas import tpu_sc as plsc

NC = 2
NS = 16
NW = NC * NS
L = 16
CW = 128
LN_EPS = 1e-5


def _silu(x):
    return x / (1.0 + jnp.exp(-x))


def _sc_mesh():
    return plsc.VectorSubcoreMesh(
        core_axis_name="c", subcore_axis_name="s", num_cores=NC, num_subcores=NS
    )


def _pre_body(nf_ref, gamma_ref, beta_ref, w1a_ref, w1b_ref, p_ref, q_ref):
    x = nf_ref[...]
    mu = jnp.mean(x, axis=-1, keepdims=True)
    xc = x - mu
    var = jnp.mean(xc * xc, axis=-1, keepdims=True)
    h = xc * lax.rsqrt(var + LN_EPS) * gamma_ref[...] + beta_ref[...]
    p_ref[...] = jnp.dot(h, w1a_ref[...], preferred_element_type=jnp.float32)
    q_ref[...] = jnp.dot(h, w1b_ref[...], preferred_element_type=jnp.float32)


def _pre_tc(nf, gamma, beta, w1a, w1b):
    n, d = nf.shape
    bn = 1000
    return pl.pallas_call(
        _pre_body,
        grid=(n // bn,),
        in_specs=[
            pl.BlockSpec((bn, d), lambda i: (i, 0)),
            pl.BlockSpec((1, d), lambda i: (0, 0)),
            pl.BlockSpec((1, d), lambda i: (0, 0)),
            pl.BlockSpec((d, d), lambda i: (0, 0)),
            pl.BlockSpec((d, d), lambda i: (0, 0)),
        ],
        out_specs=[pl.BlockSpec((bn, d), lambda i: (i, 0))] * 2,
        out_shape=[jax.ShapeDtypeStruct((n, d), jnp.float32)] * 2,
    )(nf, gamma.reshape(1, d), beta.reshape(1, d), w1a, w1b)


def _gather_sc(p, q, src, dst):
    n, d = p.shape
    e = src.shape[0]
    ew = e // NW
    c = 40
    k = ew // c
    sr = (n // NS) // 8 * 8
    zr = 16
    tail = n - NS * sr

    @functools.partial(
        pl.kernel,
        out_type=[
            jax.ShapeDtypeStruct((e, d), jnp.float32),
            jax.ShapeDtypeStruct((NC, n, CW), jnp.float32),
        ],
        mesh=_sc_mesh(),
        scratch_types=[
            pltpu.VMEM((c,), jnp.int32),
            pltpu.VMEM((c,), jnp.int32),
            pltpu.VMEM((c,), jnp.int32),
            pltpu.VMEM((c,), jnp.int32),
            pltpu.VMEM((c, d), jnp.float32),
            pltpu.VMEM((c, d), jnp.float32),
            pltpu.VMEM((c, d), jnp.float32),
            pltpu.VMEM((c, d), jnp.float32),
            pltpu.VMEM((c, CW), jnp.float32),
            pltpu.VMEM((zr, CW), jnp.float32),
            pltpu.VMEM_SHARED((n, CW), jnp.float32),
            pltpu.SemaphoreType.DMA,
            pltpu.SemaphoreType.DMA,
            pltpu.SemaphoreType.DMA,
            pltpu.SemaphoreType.DMA,
            pltpu.SemaphoreType.DMA,
            pltpu.SemaphoreType.DMA,
        ],
    )
    def gather_kernel(p_hbm, q_hbm, src_hbm, dst_hbm, g_hbm, outcnt_hbm,
                      idx_s0, idx_d0, idx_s1, idx_d1,
                      bufp0, bufq0, bufp1, bufq1,
                      ones, zbuf, cnt,
                      semp0, semq0, semc0, semp1, semq1, semc1):
        cid = lax.axis_index("c")
        sid = lax.axis_index("s")
        wid = sid * NC + cid
        base = wid * ew
        row0 = sid * sr
        zk = sr // zr

        sets = (
            (idx_s0, idx_d0, bufp0, bufq0, semp0, semq0, semc0),
            (idx_s1, idx_d1, bufp1, bufq1, semp1, semq1, semc1),
        )

        def fill_ones(i, cc):
            for j in range(CW // L):
                ones[i, pl.ds(j * L, L)] = jnp.ones((L,), jnp.float32)
            return cc

        lax.fori_loop(0, c, fill_ones, 0)

        def fill_zero(i, cc):
            for j in range(CW // L):
                zbuf[i, pl.ds(j * L, L)] = jnp.zeros((L,), jnp.float32)
            return cc

        lax.fori_loop(0, zr, fill_zero, 0)

        def zcp(z, cc):
            pltpu.sync_copy(zbuf, cnt.at[pl.ds(row0 + z * zr, zr)])
            return cc

        lax.fori_loop(0, zk, zcp, 0)

        @pl.when(sid == 0)
        def _zero_tail():
            def zcp_t(z, cc):
                pltpu.sync_copy(zbuf, cnt.at[pl.ds(NS * sr + z * zr, zr)])
                return cc
            lax.fori_loop(0, tail // zr, zcp_t, 0)

        plsc.subcore_barrier()

        def issue(kk, st):
            idx_s, idx_d, bufp, bufq, semp, semq, semc = st
            off = base + kk * c
            pltpu.sync_copy(src_hbm.at[pl.ds(off, c)], idx_s)
            pltpu.sync_copy(dst_hbm.at[pl.ds(off, c)], idx_d)
            pltpu.async_copy(p_hbm.at[idx_s], bufp, semp)
            pltpu.async_copy(q_hbm.at[idx_d], bufq, semq)
            pltpu.async_copy(ones, cnt.at[idx_s], semc, add=True)

        def process(kk, st):
            idx_s, idx_d, bufp, bufq, semp, semq, semc = st
            off = base + kk * c
            pltpu.make_async_copy(p_hbm.at[idx_s], bufp, semp).wait()
            pltpu.make_async_copy(q_hbm.at[idx_d], bufq, semq).wait()

            def add_row(i, cc):
                for j in range(d // L):
                    sl = pl.ds(j * L, L)
                    bufp[i, sl] = bufp[i, sl] + bufq[i, sl]
                return cc

            lax.fori_loop(0, c, add_row, 0)
            pltpu.sync_copy(bufp, g_hbm.at[pl.ds(off, c)])
            pltpu.make_async_copy(ones, cnt.at[idx_s], semc).wait()

        issue(0, sets[0])

        def body(t, cc):
            kk = 1 + 2 * t
            issue(kk, sets[1])
            process(kk - 1, sets[0])
            issue(kk + 1, sets[0])
            process(kk, sets[1])
            return cc

        lax.fori_loop(0, k // 2 - 1, body, 0)
        issue(k - 1, sets[1])
        process(k - 2, sets[0])
        process(k - 1, sets[1])
        plsc.subcore_barrier()

        def wb(z, cc):
            r = row0 + z * zr
            pltpu.sync_copy(cnt.at[pl.ds(r, zr)], zbuf)
            pltpu.sync_copy(zbuf, outcnt_hbm.at[cid, pl.ds(r, zr)])
            return cc

        lax.fori_loop(0, zk, wb, 0)

        @pl.when(sid == 0)
        def _wb_tail():
            def wb_t(z, cc):
                r = NS * sr + z * zr
                pltpu.sync_copy(cnt.at[pl.ds(r, zr)], zbuf)
                pltpu.sync_copy(zbuf, outcnt_hbm.at[cid, pl.ds(r, zr)])
                return cc
            lax.fori_loop(0, tail // zr, wb_t, 0)

    return gather_kernel(p, q, src, dst)


def _edge_body(g_ref, ef_ref, w1c_ref, b1_ref, w2_ref, b2_ref, out_ref):
    u = (g_ref[...]
         + jnp.dot(ef_ref[...], w1c_ref[...], preferred_element_type=jnp.float32)
         + b1_ref[...])
    u = _silu(u)
    v = jnp.dot(u, w2_ref[...], preferred_element_type=jnp.float32) + b2_ref[...]
    out_ref[...] = _silu(v)


def _edge_tc(g, ef, w1c, b1, w2, b2):
    e, d = g.shape
    ed = ef.shape[1]
    be = 2000
    return pl.pallas_call(
        _edge_body,
        grid=(e // be,),
        in_specs=[
            pl.BlockSpec((be, d), lambda i: (i, 0)),
            pl.BlockSpec((be, ed), lambda i: (i, 0)),
            pl.BlockSpec((ed, d), lambda i: (0, 0)),
            pl.BlockSpec((1, d), lambda i: (0, 0)),
            pl.BlockSpec((d, d), lambda i: (0, 0)),
            pl.BlockSpec((1, d), lambda i: (0, 0)),
        ],
        out_specs=pl.BlockSpec((be, d), lambda i: (i, 0)),
        out_shape=jax.ShapeDtypeStruct((e, d), jnp.float32),
    )(g, ef, w1c, b1.reshape(1, d), w2, b2.reshape(1, d))


def _scatter_sc(mij, src, n):
    e, d = mij.shape
    ew = e // NW
    c = 80
    k = ew // c
    sr = (n // NS) // 8 * 8
    zr = 16
    tail = n - NS * sr

    @functools.partial(
        pl.kernel,
        out_type=jax.ShapeDtypeStruct((NC, n, d), jnp.float32),
        mesh=_sc_mesh(),
        scratch_types=[
            pltpu.VMEM((c,), jnp.int32),
            pltpu.VMEM((c,), jnp.int32),
            pltpu.VMEM((c, d), jnp.float32),
            pltpu.VMEM((c, d), jnp.float32),
            pltpu.VMEM((zr, d), jnp.float32),
            pltpu.VMEM_SHARED((n, d), jnp.float32),
            pltpu.SemaphoreType.DMA,
            pltpu.SemaphoreType.DMA,
            pltpu.SemaphoreType.DMA,
            pltpu.SemaphoreType.DMA,
        ],
    )
    def scatter_kernel(mij_hbm, src_hbm, outsum_hbm,
                       idx0, idx1, buf0, buf1, zbuf, accum,
                       semr0, semi0, semr1, semi1):
        cid = lax.axis_index("c")
        sid = lax.axis_index("s")
        base = (sid * NC + cid) * ew
        row0 = sid * sr
        zk = sr // zr

        sets = ((idx0, buf0, semr0, semi0), (idx1, buf1, semr1, semi1))

        def z_row(i, cc):
            for j in range(d // L):
                zbuf[i, pl.ds(j * L, L)] = jnp.zeros((L,), jnp.float32)
            return cc

        lax.fori_loop(0, zr, z_row, 0)

        def zcp(z, cc):
            pltpu.sync_copy(zbuf, accum.at[pl.ds(row0 + z * zr, zr)])
            return cc

        lax.fori_loop(0, zk, zcp, 0)

        @pl.when(sid == 0)
        def _zero_tail():
            def zcp_t(z, cc):
                pltpu.sync_copy(zbuf, accum.at[pl.ds(NS * sr + z * zr, zr)])
                return cc
            lax.fori_loop(0, tail // zr, zcp_t, 0)

        plsc.subcore_barrier()

        def load(kk, st):
            idx, buf, semr, semi = st
            off = base + kk * c
            pltpu.async_copy(src_hbm.at[pl.ds(off, c)], idx, semi)
            pltpu.async_copy(mij_hbm.at[pl.ds(off, c)], buf, semr)

        def flush(kk, st):
            idx, buf, semr, semi = st
            off = base + kk * c
            pltpu.make_async_copy(src_hbm.at[pl.ds(off, c)], idx, semi).wait()
            pltpu.make_async_copy(mij_hbm.at[pl.ds(off, c)], buf, semr).wait()
            pltpu.sync_copy(buf, accum.at[idx], add=True)

        load(0, sets[0])

        def body(t, cc):
            kk = 1 + 2 * t
            load(kk, sets[1])
            flush(kk - 1, sets[0])
            load(kk + 1, sets[0])
            flush(kk, sets[1])
            return cc

        lax.fori_loop(0, (k - 1) // 2, body, 0)
        flush(k - 1, sets[0])
        plsc.subcore_barrier()

        def wb(z, cc):
            r = row0 + z * zr
            pltpu.sync_copy(accum.at[pl.ds(r, zr)], zbuf)
            pltpu.sync_copy(zbuf, outsum_hbm.at[cid, pl.ds(r, zr)])
            return cc

        lax.fori_loop(0, zk, wb, 0)

        @pl.when(sid == 0)
        def _wb_tail():
            def wb_t(z, cc):
                r = NS * sr + z * zr
                pltpu.sync_copy(accum.at[pl.ds(r, zr)], zbuf)
                pltpu.sync_copy(zbuf, outsum_hbm.at[cid, pl.ds(r, zr)])
                return cc
            lax.fori_loop(0, tail // zr, wb_t, 0)

    return scatter_kernel(mij, src)


def _final_body(nf_ref, ms0_ref, ms1_ref, cn0_ref, cn1_ref,
                a1a_ref, a1b_ref, c1_ref, a2_ref, c2_ref, out_ref):
    x = nf_ref[...]
    cnt = cn0_ref[...][:, :1] + cn1_ref[...][:, :1]
    msg = (ms0_ref[...] + ms1_ref[...]) / jnp.maximum(cnt, 1.0)
    u = (jnp.dot(x, a1a_ref[...], preferred_element_type=jnp.float32)
         + jnp.dot(msg, a1b_ref[...], preferred_element_type=jnp.float32)
         + c1_ref[...])
    u = _silu(u)
    v = jnp.dot(u, a2_ref[...], preferred_element_type=jnp.float32) + c2_ref[...]
    out_ref[...] = x + _silu(v)


def _final_tc(nf, ms0, ms1, cn0, cn1, a1a, a1b, c1, a2, c2):
    n, d = nf.shape
    bn = 1000
    return pl.pallas_call(
        _final_body,
        grid=(n // bn,),
        in_specs=[pl.BlockSpec((bn, d), lambda i: (i, 0))] * 3 + [
            pl.BlockSpec((bn, CW), lambda i: (i, 0)),
            pl.BlockSpec((bn, CW), lambda i: (i, 0)),
        ] + [
            pl.BlockSpec((d, d), lambda i: (0, 0)),
            pl.BlockSpec((d, d), lambda i: (0, 0)),
            pl.BlockSpec((1, d), lambda i: (0, 0)),
            pl.BlockSpec((d, d), lambda i: (0, 0)),
            pl.BlockSpec((1, d), lambda i: (0, 0)),
        ],
        out_specs=pl.BlockSpec((bn, d), lambda i: (i, 0)),
        out_shape=jax.ShapeDtypeStruct((n, d), jnp.float32),
    )(nf, ms0, ms1, cn0, cn1, a1a, a1b, c1.reshape(1, d), a2, c2.reshape(1, d))


def kernel(node_features, edge_features, edge_index, edge2graph,
           W1, b1, W2, b2, A1, c1, A2, c2, gamma, beta):
    n, d = node_features.shape
    src = edge_index[0]
    dst = edge_index[1]
    w1a, w1b, w1c = W1[:d], W1[d:2 * d], W1[2 * d:]
    a1a, a1b = A1[:d], A1[d:]

    p, q = _pre_tc(node_features, gamma, beta, w1a, w1b)
    g, cnth = _gather_sc(p, q, src, dst)
    mij = _edge_tc(g, edge_features, w1c, b1, W2, b2)
    msum = _scatter_sc(mij, src, n)
    return _final_tc(node_features, msum[0], msum[1], cnth[0], cnth[1],
                     a1a, a1b, c1, A2, c2)
